# baseline (speedup 1.0000x reference)
import functools

import jax
import jax.numpy as jnp
from jax import lax
from jax.experimental import pallas as pl
from jax.experimental.pallas import tpu as pltpu
from jax.experimental.pallas import tpu_sc as plsc

_DIM = 32
_CB = 8192
_N = 4096
_BN = 1024


def _nextafter_pos(c):
    return lax.bitcast_convert_type(
        lax.bitcast_convert_type(c, jnp.int32) + 1, jnp.float32)


def _argmin_body(x_ref, e_ref, a2_ref, b2_ref, idsf_ref, out_ref):
    x = x_ref[...]
    e = e_ref[...]
    a2 = a2_ref[...]
    b2 = b2_ref[...]
    idsf = idsf_ref[...]
    ab = lax.dot_general(x, e, (((1,), (1,)), ((), ())),
                         preferred_element_type=jnp.float32)
    d2 = a2 + b2 - 2.0 * ab
    m2 = jnp.maximum(jnp.min(d2, axis=1, keepdims=True), 0.0)
    s = jnp.sqrt(m2)
    B = m2
    c = m2
    for _ in range(4):
        c = _nextafter_pos(c)
        B = jnp.where(jnp.sqrt(c) == s, c, B)
    cand = jnp.where(d2 <= B, idsf, jnp.float32(2 * _CB))
    idxf = jnp.min(cand, axis=1, keepdims=True)
    out_ref[...] = idxf.astype(jnp.int32)


def _tc_argmin(xf, et, a2, b2):
    n = xf.shape[0]
    grid = (n // _BN,)
    return pl.pallas_call(
        _argmin_body,
        grid=grid,
        in_specs=[
            pl.BlockSpec((_BN, _DIM), lambda i: (i, 0)),
            pl.BlockSpec((_CB, _DIM), lambda i: (0, 0)),
            pl.BlockSpec((_BN, 1), lambda i: (i, 0)),
            pl.BlockSpec((1, _CB), lambda i: (0, 0)),
            pl.BlockSpec((1, _CB), lambda i: (0, 0)),
        ],
        out_specs=pl.BlockSpec((_BN, 1), lambda i: (i, 0)),
        out_shape=jax.ShapeDtypeStruct((n, 1), jnp.int32),
    )(xf, et, a2, b2, jnp.arange(_CB, dtype=jnp.float32)[None, :])


_NW = 32


def _sc_gather_body(bpw, table_hbm, idx_hbm, out_hbm, idx_v, rows_v, sem):
    wid = lax.axis_index("s") * 2 + lax.axis_index("c")
    base = wid * bpw
    pltpu.sync_copy(idx_hbm.at[pl.ds(base, bpw)], idx_v)
    pltpu.async_copy(table_hbm.at[idx_v], rows_v, sem).wait()
    pltpu.sync_copy(rows_v, out_hbm.at[pl.ds(base, bpw)])


def _sc_gather(table, idx):
    n = idx.shape[0]
    bpw = n // _NW
    mesh = plsc.VectorSubcoreMesh(core_axis_name="c", subcore_axis_name="s")
    k = functools.partial(
        pl.kernel,
        mesh=mesh,
        out_type=jax.ShapeDtypeStruct((n, _DIM), jnp.float32),
        scratch_types=[
            pltpu.VMEM((bpw,), jnp.int32),
            pltpu.VMEM((bpw, _DIM), jnp.float32),
            pltpu.SemaphoreType.DMA,
        ],
        compiler_params=pltpu.CompilerParams(use_tc_tiling_on_sc=False),
    )(functools.partial(_sc_gather_body, bpw))
    return k(table, idx)


def kernel(x, embed):
    shape = x.shape
    xf = x.reshape(-1, _DIM)
    a2 = jnp.sum(xf * xf, axis=1, keepdims=True)
    b2 = jnp.sum(embed * embed, axis=1)[None, :]
    half = _N // 2
    ind_a = _tc_argmin(xf[:half], embed, a2[:half], b2).reshape(-1)
    q_a = _sc_gather(embed, ind_a)
    ind_b = _tc_argmin(xf[half:], embed, a2[half:], b2).reshape(-1)
    q_b = _sc_gather(embed, ind_b)
    quantize = jnp.concatenate([q_a, q_b], axis=0)
    ind = jnp.concatenate([ind_a, ind_b], axis=0)
    return quantize.reshape(shape), ind.reshape(shape[:-1])


# speedup vs baseline: 1.1183x; 1.1183x over previous
import functools

import jax
import jax.numpy as jnp
from jax import lax
from jax.experimental import pallas as pl
from jax.experimental.pallas import tpu as pltpu
from jax.experimental.pallas import tpu_sc as plsc

_DIM = 32
_CB = 8192
_N = 4096
_BN = 1024


def _nextafter_pos(c):
    return lax.bitcast_convert_type(
        lax.bitcast_convert_type(c, jnp.int32) + 1, jnp.float32)


def _argmin_body(x_ref, e_ref, a2_ref, b2_ref, idsf_ref, out_ref):
    x = x_ref[...]
    e = e_ref[...]
    a2 = a2_ref[...]
    b2 = b2_ref[...]
    idsf = idsf_ref[...]
    ab = lax.dot_general(x, e, (((1,), (1,)), ((), ())),
                         preferred_element_type=jnp.float32)
    d2 = a2 + b2 - 2.0 * ab
    m2 = jnp.maximum(jnp.min(d2, axis=1, keepdims=True), 0.0)
    s = jnp.sqrt(m2)
    B = m2
    c = m2
    for _ in range(4):
        c = _nextafter_pos(c)
        B = jnp.where(jnp.sqrt(c) == s, c, B)
    cand = jnp.where(d2 <= B, idsf, jnp.float32(2 * _CB))
    idxf = jnp.min(cand, axis=1, keepdims=True)
    out_ref[...] = idxf.astype(jnp.int32)


def _tc_argmin(xf, et, a2, b2):
    n = xf.shape[0]
    grid = (n // _BN,)
    return pl.pallas_call(
        _argmin_body,
        grid=grid,
        in_specs=[
            pl.BlockSpec((_BN, _DIM), lambda i: (i, 0)),
            pl.BlockSpec((_CB, _DIM), lambda i: (0, 0)),
            pl.BlockSpec((_BN, 1), lambda i: (i, 0)),
            pl.BlockSpec((1, _CB), lambda i: (0, 0)),
            pl.BlockSpec((1, _CB), lambda i: (0, 0)),
        ],
        out_specs=pl.BlockSpec((_BN, 1), lambda i: (i, 0)),
        out_shape=jax.ShapeDtypeStruct((n, 1), jnp.int32),
    )(xf, et, a2, b2, jnp.arange(_CB, dtype=jnp.float32)[None, :])


_NW = 32


def _sc_gather_body(bpw, table_hbm, idx_hbm, out_hbm, idx_v, rows_v, sem):
    wid = lax.axis_index("s") * 2 + lax.axis_index("c")
    base = wid * bpw
    pltpu.sync_copy(idx_hbm.at[pl.ds(base, bpw)], idx_v)
    pltpu.async_copy(table_hbm.at[idx_v], rows_v, sem).wait()
    pltpu.sync_copy(rows_v, out_hbm.at[pl.ds(base, bpw)])


def _sc_gather(table, idx):
    n = idx.shape[0]
    bpw = n // _NW
    mesh = plsc.VectorSubcoreMesh(core_axis_name="c", subcore_axis_name="s")
    k = functools.partial(
        pl.kernel,
        mesh=mesh,
        out_type=jax.ShapeDtypeStruct((n, _DIM), jnp.float32),
        scratch_types=[
            pltpu.VMEM((bpw,), jnp.int32),
            pltpu.VMEM((bpw, _DIM), jnp.float32),
            pltpu.SemaphoreType.DMA,
        ],
        compiler_params=pltpu.CompilerParams(use_tc_tiling_on_sc=False),
    )(functools.partial(_sc_gather_body, bpw))
    return k(table, idx)


def kernel(x, embed):
    shape = x.shape
    xf = x.reshape(-1, _DIM)
    a2 = jnp.sum(xf * xf, axis=1, keepdims=True)
    b2 = jnp.sum(embed * embed, axis=1)[None, :]
    ind = _tc_argmin(xf, embed, a2, b2).reshape(-1)
    quantize = _sc_gather(embed, ind)
    return quantize.reshape(shape), ind.reshape(shape[:-1])
